# Initial kernel scaffold; baseline (speedup 1.0000x reference)
import jax
import jax.numpy as jnp
from jax.experimental import pallas as pl


def kernel(hidden_states, weight):
    raise NotImplementedError("write your pallas kernel here")


# speedup vs baseline: 1.1565x; 1.1565x over previous
import jax
import jax.numpy as jnp
from jax.experimental import pallas as pl
from jax.experimental.pallas import tpu as pltpu

E = 64
K = 8
ALPHA = 0.01
ROWS = 512


def _gate_kernel(x_ref, w_ref, idx_ref, val_ref, acc_ref):
    x = x_ref[...]
    w = w_ref[...]
    logits = jax.lax.dot_general(
        x, w, (((1,), (1,)), ((), ())), preferred_element_type=jnp.float32
    )
    m = jnp.max(logits, axis=-1, keepdims=True)
    e = jnp.exp(logits - m)
    denom = jnp.sum(e, axis=-1, keepdims=True)
    scores = e / denom

    @pl.when(pl.program_id(0) == 0)
    def _init():
        acc_ref[0, 0] = 0.0

    acc_ref[0, 0] += jnp.sum(scores)

    iota = jax.lax.broadcasted_iota(jnp.int32, scores.shape, 1)
    work = scores
    vals = []
    idxs = []
    for _ in range(K):
        mk = jnp.max(work, axis=-1, keepdims=True)
        sel = jnp.min(jnp.where(work == mk, iota, E), axis=-1, keepdims=True)
        vals.append(mk)
        idxs.append(sel)
        work = jnp.where(iota == sel, -1.0, work)
    val_ref[...] = jnp.concatenate(vals, axis=1)
    idx_ref[...] = jnp.concatenate(idxs, axis=1)


def kernel(hidden_states, weight):
    b, s, h = hidden_states.shape
    n = b * s
    hs = hidden_states.reshape(n, h)
    idx, val, acc = pl.pallas_call(
        _gate_kernel,
        grid=(n // ROWS,),
        in_specs=[
            pl.BlockSpec((ROWS, h), lambda i: (i, 0)),
            pl.BlockSpec((E, h), lambda i: (0, 0)),
        ],
        out_specs=[
            pl.BlockSpec((ROWS, K), lambda i: (i, 0)),
            pl.BlockSpec((ROWS, K), lambda i: (i, 0)),
            pl.BlockSpec(memory_space=pltpu.SMEM),
        ],
        out_shape=[
            jax.ShapeDtypeStruct((n, K), jnp.int32),
            jax.ShapeDtypeStruct((n, K), jnp.float32),
            jax.ShapeDtypeStruct((1, 1), jnp.float32),
        ],
    )(hs, weight)
    aux_loss = acc[0, 0] * (ALPHA / n)
    return idx, val, aux_loss


# speedup vs baseline: 2.1605x; 1.8682x over previous
import jax
import jax.numpy as jnp
from jax.experimental import pallas as pl
from jax.experimental.pallas import tpu as pltpu

E = 64
K = 8
ALPHA = 0.01
ROWS = 512


def _gate_kernel(x_ref, w_ref, idx_ref, val_ref, acc_ref):
    x = x_ref[...]
    w = w_ref[...]
    logits = jax.lax.dot_general(
        w, x, (((1,), (1,)), ((), ())), preferred_element_type=jnp.float32
    )
    m = jnp.max(logits, axis=0, keepdims=True)
    e = jnp.exp(logits - m)
    denom = jnp.sum(e, axis=0, keepdims=True)
    scores = e / denom

    @pl.when(pl.program_id(0) == 0)
    def _init():
        acc_ref[0, 0] = 0.0

    acc_ref[0, 0] += jnp.sum(scores)

    iota = jax.lax.broadcasted_iota(jnp.int32, scores.shape, 0)
    work = scores
    vals = []
    idxs = []
    for _ in range(K):
        mk = jnp.max(work, axis=0, keepdims=True)
        sel = jnp.min(jnp.where(work == mk, iota, E), axis=0, keepdims=True)
        vals.append(mk)
        idxs.append(sel)
        work = jnp.where(iota == sel, -1.0, work)
    val_ref[...] = jnp.concatenate(vals, axis=0)
    idx_ref[...] = jnp.concatenate(idxs, axis=0)


def kernel(hidden_states, weight):
    b, s, h = hidden_states.shape
    n = b * s
    hs = hidden_states.reshape(n, h)
    idx_t, val_t, acc = pl.pallas_call(
        _gate_kernel,
        grid=(n // ROWS,),
        in_specs=[
            pl.BlockSpec((ROWS, h), lambda i: (i, 0)),
            pl.BlockSpec((E, h), lambda i: (0, 0)),
        ],
        out_specs=[
            pl.BlockSpec((K, ROWS), lambda i: (0, i)),
            pl.BlockSpec((K, ROWS), lambda i: (0, i)),
            pl.BlockSpec(memory_space=pltpu.SMEM),
        ],
        out_shape=[
            jax.ShapeDtypeStruct((K, n), jnp.int32),
            jax.ShapeDtypeStruct((K, n), jnp.float32),
            jax.ShapeDtypeStruct((1, 1), jnp.float32),
        ],
    )(hs, weight)
    aux_loss = acc[0, 0] * (ALPHA / n)
    return idx_t.T, val_t.T, aux_loss
